# Initial kernel scaffold; baseline (speedup 1.0000x reference)
import jax
import jax.numpy as jnp
from jax.experimental import pallas as pl


def kernel(input, mipmap_0, mipmap_1, mipmap_2, mipmap_3):
    raise NotImplementedError("write your pallas kernel here")


# speedup vs baseline: 44.4405x; 44.4405x over previous
import functools

import jax
import jax.numpy as jnp
from jax import lax
from jax.experimental import pallas as pl
from jax.experimental.pallas import tpu as pltpu
from jax.experimental.pallas import tpu_sc as plsc

_SIZES = (1024, 512, 256, 128)
_C = 16
_B = 4
_HW = 512
_P = _B * _HW * _HW
_NW = 32
_PPW = _P // _NW
_CHUNK = 128
_NCHUNK = _PPW // _CHUNK
_NG = 16


def _sc_sample(u, v, t0, t1, t2, t3):
    mesh = plsc.VectorSubcoreMesh(core_axis_name="c", subcore_axis_name="s")

    @functools.partial(
        pl.kernel,
        mesh=mesh,
        out_type=jax.ShapeDtypeStruct((_P, _C), jnp.float32),
        compiler_params=pltpu.CompilerParams(use_tc_tiling_on_sc=False),
        scratch_types=[
            pltpu.VMEM((_CHUNK,), jnp.float32),
            pltpu.VMEM((_CHUNK,), jnp.float32),
            pltpu.VMEM((_NG, _CHUNK), jnp.int32),
            pltpu.VMEM((_NG, _CHUNK), jnp.float32),
            pltpu.VMEM((_NG, _CHUNK, _C), jnp.float32),
            pltpu.VMEM((_CHUNK, _C), jnp.float32),
            pltpu.SemaphoreType.DMA,
        ],
    )
    def body(u_hbm, v_hbm, t0_hbm, t1_hbm, t2_hbm, t3_hbm, out_hbm,
             u_v, v_v, idx_v, w_v, rows_v, o_v, sem):
        tabs = (t0_hbm, t1_hbm, t2_hbm, t3_hbm)
        wid = lax.axis_index("s") * 2 + lax.axis_index("c")

        def chunk_body(g, carry):
            base = wid * _PPW + g * _CHUNK
            pltpu.sync_copy(u_hbm.at[pl.ds(base, _CHUNK)], u_v)
            pltpu.sync_copy(v_hbm.at[pl.ds(base, _CHUNK)], v_v)

            def grp_body(gi, c2):
                sl = pl.ds(gi * 16, 16)
                uu = u_v[sl]
                vv = v_v[sl]
                for li, s in enumerate(_SIZES):
                    sp = s + 2
                    ix = ((2.0 * uu - 1.0 + 1.0) * s - 1.0) * 0.5
                    iy = ((2.0 * vv - 1.0 + 1.0) * s - 1.0) * 0.5
                    x0 = (ix + 1.0).astype(jnp.int32)
                    y0 = (iy + 1.0).astype(jnp.int32)
                    x0 = jnp.minimum(jnp.maximum(x0, 0), s)
                    y0 = jnp.minimum(jnp.maximum(y0, 0), s)
                    fx = ix - (x0.astype(jnp.float32) - 1.0)
                    fy = iy - (y0.astype(jnp.float32) - 1.0)
                    i00 = y0 * sp + x0
                    w1y = fy
                    w0y = 1.0 - fy
                    w1x = fx
                    w0x = 1.0 - fx
                    idx_v[li * 4 + 0, sl] = i00
                    idx_v[li * 4 + 1, sl] = i00 + 1
                    idx_v[li * 4 + 2, sl] = i00 + sp
                    idx_v[li * 4 + 3, sl] = i00 + sp + 1
                    w_v[li * 4 + 0, sl] = w0x * w0y
                    w_v[li * 4 + 1, sl] = w1x * w0y
                    w_v[li * 4 + 2, sl] = w0x * w1y
                    w_v[li * 4 + 3, sl] = w1x * w1y
                return c2

            lax.fori_loop(0, _CHUNK // 16, grp_body, 0)

            copies = []
            for li in range(4):
                for c in range(4):
                    k = li * 4 + c
                    copies.append(pltpu.async_copy(
                        tabs[li].at[idx_v.at[k]], rows_v.at[k], sem))
            for cp in copies:
                cp.wait()

            def wgrp_body(gi, c3):
                sl = pl.ds(gi * 16, 16)
                wk = [w_v[k, sl] for k in range(_NG)]
                for j in range(16):
                    p = gi * 16 + j
                    acc = wk[0][j] * rows_v[0, p]
                    for k in range(1, _NG):
                        acc = acc + wk[k][j] * rows_v[k, p]
                    o_v[p] = acc
                return c3

            lax.fori_loop(0, _CHUNK // 16, wgrp_body, 0)
            pltpu.sync_copy(o_v, out_hbm.at[pl.ds(base, _CHUNK)])
            return carry

        lax.fori_loop(0, _NCHUNK, chunk_body, 0)

    return body(u, v, t0, t1, t2, t3)


def _prep_table(m, s):
    t = jnp.transpose(m[0], (1, 2, 0))
    t = jnp.pad(t, ((1, 1), (1, 1), (0, 0)))
    return t.reshape((s + 2) * (s + 2), _C)


def kernel(input, mipmap_0, mipmap_1, mipmap_2, mipmap_3):
    u = input[..., 0].reshape(_P)
    v = input[..., 1].reshape(_P)
    tables = [_prep_table(m, s)
              for m, s in zip((mipmap_0, mipmap_1, mipmap_2, mipmap_3), _SIZES)]
    out = _sc_sample(u, v, *tables)
    return out.reshape(_B, _HW, _HW, _C).transpose(0, 3, 1, 2)


# speedup vs baseline: 90.7847x; 2.0428x over previous
import functools

import jax
import jax.numpy as jnp
from jax import lax
from jax.experimental import pallas as pl
from jax.experimental.pallas import tpu as pltpu
from jax.experimental.pallas import tpu_sc as plsc

_SIZES = (1024, 512, 256, 128)
_C = 16
_B = 4
_HW = 512
_P = _B * _HW * _HW
_NW = 32
_PPW = _P // _NW
_CHUNK = 128
_NCHUNK = _PPW // _CHUNK
_NG = 16


def _sc_sample(u, v, t0, t1, t2, t3):
    mesh = plsc.VectorSubcoreMesh(core_axis_name="c", subcore_axis_name="s")

    @functools.partial(
        pl.kernel,
        mesh=mesh,
        out_type=jax.ShapeDtypeStruct((_P, _C), jnp.float32),
        compiler_params=pltpu.CompilerParams(use_tc_tiling_on_sc=False),
        scratch_types=[
            pltpu.VMEM((_CHUNK,), jnp.float32),
            pltpu.VMEM((_CHUNK,), jnp.float32),
            pltpu.VMEM((_NG, _CHUNK), jnp.int32),
            pltpu.VMEM((_NG, _CHUNK), jnp.float32),
            pltpu.VMEM((_NG, _CHUNK, _C), jnp.float32),
            pltpu.VMEM((_CHUNK, _C), jnp.float32),
            pltpu.SemaphoreType.DMA,
        ],
    )
    def body(u_hbm, v_hbm, t0_hbm, t1_hbm, t2_hbm, t3_hbm, out_hbm,
             u_v, v_v, idx_v, w_v, rows_v, o_v, sem):
        tabs = (t0_hbm, t1_hbm, t2_hbm, t3_hbm)
        wid = lax.axis_index("s") * 2 + lax.axis_index("c")

        def chunk_body(g, carry):
            base = wid * _PPW + g * _CHUNK
            pltpu.sync_copy(u_hbm.at[pl.ds(base, _CHUNK)], u_v)
            pltpu.sync_copy(v_hbm.at[pl.ds(base, _CHUNK)], v_v)

            def grp_body(gi, c2):
                sl = pl.ds(gi * 16, 16)
                uu = u_v[sl]
                vv = v_v[sl]
                for li, s in enumerate(_SIZES):
                    ix = ((2.0 * uu - 1.0 + 1.0) * s - 1.0) * 0.5
                    iy = ((2.0 * vv - 1.0 + 1.0) * s - 1.0) * 0.5
                    x0i = (ix + 1.0).astype(jnp.int32)
                    y0i = (iy + 1.0).astype(jnp.int32)
                    fx = ix - (x0i.astype(jnp.float32) - 1.0)
                    fy = iy - (y0i.astype(jnp.float32) - 1.0)
                    xc0 = jnp.maximum(x0i - 1, 0)
                    xc1 = jnp.minimum(jnp.maximum(x0i, 0), s - 1)
                    yc0 = jnp.maximum(y0i - 1, 0)
                    yc1 = jnp.minimum(jnp.maximum(y0i, 0), s - 1)
                    w0x = jnp.where(x0i >= 1, 1.0 - fx, 0.0)
                    w1x = jnp.where(x0i <= s - 1, fx, 0.0)
                    w0y = jnp.where(y0i >= 1, 1.0 - fy, 0.0)
                    w1y = jnp.where(y0i <= s - 1, fy, 0.0)
                    r0 = yc0 * s
                    r1 = yc1 * s
                    idx_v[li * 4 + 0, sl] = r0 + xc0
                    idx_v[li * 4 + 1, sl] = r0 + xc1
                    idx_v[li * 4 + 2, sl] = r1 + xc0
                    idx_v[li * 4 + 3, sl] = r1 + xc1
                    w_v[li * 4 + 0, sl] = w0x * w0y
                    w_v[li * 4 + 1, sl] = w1x * w0y
                    w_v[li * 4 + 2, sl] = w0x * w1y
                    w_v[li * 4 + 3, sl] = w1x * w1y
                return c2

            lax.fori_loop(0, _CHUNK // 16, grp_body, 0)

            copies = []
            for li in range(4):
                for c in range(4):
                    k = li * 4 + c
                    copies.append(pltpu.async_copy(
                        tabs[li].at[idx_v.at[k]], rows_v.at[k], sem))
            for cp in copies:
                cp.wait()

            def wgrp_body(gi, c3):
                sl = pl.ds(gi * 16, 16)
                wk = [w_v[k, sl] for k in range(_NG)]
                for j in range(16):
                    p = gi * 16 + j
                    acc = wk[0][j] * rows_v[0, p]
                    for k in range(1, _NG):
                        acc = acc + wk[k][j] * rows_v[k, p]
                    o_v[p] = acc
                return c3

            lax.fori_loop(0, _CHUNK // 16, wgrp_body, 0)
            pltpu.sync_copy(o_v, out_hbm.at[pl.ds(base, _CHUNK)])
            return carry

        lax.fori_loop(0, _NCHUNK, chunk_body, 0)

    return body(u, v, t0, t1, t2, t3)


def _prep_table(m, s):
    return jnp.transpose(m.reshape(_C, s * s))


def kernel(input, mipmap_0, mipmap_1, mipmap_2, mipmap_3):
    u = input[..., 0].reshape(_P)
    v = input[..., 1].reshape(_P)
    tables = [_prep_table(m, s)
              for m, s in zip((mipmap_0, mipmap_1, mipmap_2, mipmap_3), _SIZES)]
    out = _sc_sample(u, v, *tables)
    return out.reshape(_B, _HW, _HW, _C).transpose(0, 3, 1, 2)


# speedup vs baseline: 105.8849x; 1.1663x over previous
import functools

import jax
import jax.numpy as jnp
from jax import lax
from jax.experimental import pallas as pl
from jax.experimental.pallas import tpu as pltpu
from jax.experimental.pallas import tpu_sc as plsc

_SIZES = (1024, 512, 256, 128)
_C = 16
_B = 4
_HW = 512
_P = _B * _HW * _HW
_NW = 32
_PPW = _P // _NW
_CHUNK = 128
_NCHUNK = _PPW // _CHUNK
_NG = 16


def _sc_sample(u, v, t0, t1, t2, t3):
    mesh = plsc.VectorSubcoreMesh(core_axis_name="c", subcore_axis_name="s")

    @functools.partial(
        pl.kernel,
        mesh=mesh,
        out_type=jax.ShapeDtypeStruct((_P, _C), jnp.float32),
        compiler_params=pltpu.CompilerParams(use_tc_tiling_on_sc=False),
        scratch_types=[
            pltpu.VMEM((_CHUNK,), jnp.float32),
            pltpu.VMEM((_CHUNK,), jnp.float32),
            pltpu.VMEM((_NG, _CHUNK), jnp.int32),
            pltpu.VMEM((_NG, _CHUNK), jnp.int32),
            pltpu.VMEM((_NG, _CHUNK), jnp.float32),
            pltpu.VMEM((_NG, _CHUNK), jnp.float32),
            pltpu.VMEM((_NG * _CHUNK, _C), jnp.float32),
            pltpu.VMEM((_NG * _CHUNK, _C), jnp.float32),
            pltpu.VMEM((_CHUNK, _C), jnp.float32),
            pltpu.SemaphoreType.DMA,
            pltpu.SemaphoreType.DMA,
        ],
    )
    def body(u_hbm, v_hbm, t0_hbm, t1_hbm, t2_hbm, t3_hbm, out_hbm,
             u_v, v_v, idxA, idxB, wA, wB, rowsA, rowsB, o_v, semA, semB):
        tabs = (t0_hbm, t1_hbm, t2_hbm, t3_hbm)
        wid = lax.axis_index("s") * 2 + lax.axis_index("c")
        wbase = wid * _PPW

        def fire(g, idx_v, w_v, rows_v, sem):
            base = wbase + g * _CHUNK
            pltpu.sync_copy(u_hbm.at[pl.ds(base, _CHUNK)], u_v)
            pltpu.sync_copy(v_hbm.at[pl.ds(base, _CHUNK)], v_v)

            def grp_body(gi, c2):
                sl = pl.ds(gi * 16, 16)
                uu = u_v[sl]
                vv = v_v[sl]
                for li, s in enumerate(_SIZES):
                    ix = ((2.0 * uu - 1.0 + 1.0) * s - 1.0) * 0.5
                    iy = ((2.0 * vv - 1.0 + 1.0) * s - 1.0) * 0.5
                    x0i = (ix + 1.0).astype(jnp.int32)
                    y0i = (iy + 1.0).astype(jnp.int32)
                    fx = ix - (x0i.astype(jnp.float32) - 1.0)
                    fy = iy - (y0i.astype(jnp.float32) - 1.0)
                    xc0 = jnp.maximum(x0i - 1, 0)
                    xc1 = jnp.minimum(jnp.maximum(x0i, 0), s - 1)
                    yc0 = jnp.maximum(y0i - 1, 0)
                    yc1 = jnp.minimum(jnp.maximum(y0i, 0), s - 1)
                    w0x = jnp.where(x0i >= 1, 1.0 - fx, 0.0)
                    w1x = jnp.where(x0i <= s - 1, fx, 0.0)
                    w0y = jnp.where(y0i >= 1, 1.0 - fy, 0.0)
                    w1y = jnp.where(y0i <= s - 1, fy, 0.0)
                    r0 = yc0 * s
                    r1 = yc1 * s
                    idx_v[li * 4 + 0, sl] = r0 + xc0
                    idx_v[li * 4 + 1, sl] = r0 + xc1
                    idx_v[li * 4 + 2, sl] = r1 + xc0
                    idx_v[li * 4 + 3, sl] = r1 + xc1
                    w_v[li * 4 + 0, sl] = w0x * w0y
                    w_v[li * 4 + 1, sl] = w1x * w0y
                    w_v[li * 4 + 2, sl] = w0x * w1y
                    w_v[li * 4 + 3, sl] = w1x * w1y
                return c2

            lax.fori_loop(0, _CHUNK // 16, grp_body, 0)
            for li in range(4):
                for c in range(4):
                    k = li * 4 + c
                    pltpu.async_copy(
                        tabs[li].at[idx_v.at[k]],
                        rows_v.at[pl.ds(k * _CHUNK, _CHUNK)], sem)

        def process(g, w_v, rows_v, sem):
            pltpu.make_async_copy(
                out_hbm.at[pl.ds(0, _NG * _CHUNK)], rows_v, sem).wait()

            def wgrp_body(gi, c3):
                sl = pl.ds(gi * 16, 16)
                wk = [w_v[k, sl] for k in range(_NG)]
                for j in range(16):
                    p = gi * 16 + j
                    acc = wk[0][j] * rows_v[p]
                    for k in range(1, _NG):
                        acc = acc + wk[k][j] * rows_v[k * _CHUNK + p]
                    o_v[p] = acc
                return c3

            lax.fori_loop(0, _CHUNK // 16, wgrp_body, 0)
            base = wbase + g * _CHUNK
            pltpu.sync_copy(o_v, out_hbm.at[pl.ds(base, _CHUNK)])

        fire(0, idxA, wA, rowsA, semA)

        def pair_body(i, carry):
            g0 = 2 * i
            fire(g0 + 1, idxB, wB, rowsB, semB)
            process(g0, wA, rowsA, semA)

            @pl.when(i < _NCHUNK // 2 - 1)
            def _():
                fire(g0 + 2, idxA, wA, rowsA, semA)

            process(g0 + 1, wB, rowsB, semB)
            return carry

        lax.fori_loop(0, _NCHUNK // 2, pair_body, 0)

    return body(u, v, t0, t1, t2, t3)


def _prep_table(m, s):
    return jnp.transpose(m.reshape(_C, s * s))


def kernel(input, mipmap_0, mipmap_1, mipmap_2, mipmap_3):
    u = input[..., 0].reshape(_P)
    v = input[..., 1].reshape(_P)
    tables = [_prep_table(m, s)
              for m, s in zip((mipmap_0, mipmap_1, mipmap_2, mipmap_3), _SIZES)]
    out = _sc_sample(u, v, *tables)
    return out.reshape(_B, _HW, _HW, _C).transpose(0, 3, 1, 2)
